# baseline (speedup 1.0000x reference)
import jax
import jax.numpy as jnp
from jax import lax
from jax.experimental import pallas as pl
from jax.experimental.pallas import tpu as pltpu
from jax.experimental.pallas import tpu_sc as plsc

N = 10000
E = 320000
D = 128

NC = 2
NS = 16
NW = NC * NS
CE = 128
NCHUNK = E // CE
MAXIT = -(-NCHUNK // NW)
NBUF = 2
PAIRS = -(-MAXIT // NBUF)
RPT = 640
RPT_LAST = N - (NS - 1) * RPT
ZRPT = N // NS


def _sc_scatter(h_edge, edge_index, z640, z1d, history_deg, history_neigh):
  mesh = plsc.VectorSubcoreMesh(core_axis_name="c", subcore_axis_name="s")

  def body(h_edge_hbm, ei_hbm, z640_hbm, z1d_hbm, hd_hbm, hn_hbm,
           feat_out, cnt_out,
           stage, idxbuf, cnt_local, acc_feat,
           fsem0, fsem1, isem0, isem1, ssem):
    c = lax.axis_index("c")
    s = lax.axis_index("s")
    wid = c * NS + s
    my_rows = jnp.where(s == NS - 1, RPT_LAST, RPT)
    sbase = s * RPT
    fsems = (fsem0, fsem1)
    isems = (isem0, isem1)

    @pl.when(c == 0)
    def _():
      pltpu.sync_copy(hn_hbm.at[pl.ds(sbase, my_rows)],
                      acc_feat.at[pl.ds(sbase, my_rows)])

    @pl.when(c != 0)
    def _():
      pltpu.sync_copy(z640_hbm.at[pl.ds(0, my_rows)],
                      acc_feat.at[pl.ds(sbase, my_rows)])

    @pl.when(wid == 0)
    def _():
      pltpu.sync_copy(hd_hbm, cnt_local)

    @pl.when(wid != 0)
    def _():
      pltpu.sync_copy(z1d_hbm, cnt_local)

    ones_vec = jnp.ones((16,), jnp.float32)

    def issue(it, b):
      k = wid + it * NW

      @pl.when(k < NCHUNK)
      def _():
        pltpu.async_copy(h_edge_hbm.at[pl.ds(k * CE, CE)], stage.at[b],
                         fsems[b])
        pltpu.async_copy(ei_hbm.at[1, pl.ds(k * CE, CE)], idxbuf.at[b],
                         isems[b])

    for b in range(NBUF):
      issue(b, b)

    plsc.subcore_barrier()

    def wait_scatter(it, b):
      k = wid + it * NW

      @pl.when(k < NCHUNK)
      def _():
        pltpu.make_async_copy(h_edge_hbm.at[pl.ds(0, CE)], stage.at[b],
                              fsems[b]).wait()
        pltpu.make_async_copy(ei_hbm.at[1, pl.ds(0, CE)], idxbuf.at[b],
                              isems[b]).wait()
        cd = pltpu.async_copy(stage.at[b], acc_feat.at[idxbuf.at[b]], ssem,
                              add=True)
        for v in range(CE // 16):
          idx16 = idxbuf[b, pl.ds(v * 16, 16)]
          plsc.addupdate_scatter(cnt_local, [idx16], ones_vec)
        cd.wait()

    def pair_body(i, carry):
      for b in range(NBUF):
        it = i * NBUF + b
        wait_scatter(it, b)
        issue(it + NBUF, b)
      return carry

    lax.fori_loop(0, PAIRS, pair_body, 0)
    plsc.subcore_barrier()

    cd1 = pltpu.async_copy(acc_feat.at[pl.ds(sbase, my_rows)],
                           feat_out.at[c, pl.ds(sbase, my_rows)], fsem0)
    cd2 = pltpu.async_copy(cnt_local, cnt_out.at[pl.ds(wid * N, N)], isem0)
    cd1.wait()
    cd2.wait()

  f = pl.kernel(
      body,
      out_type=(jax.ShapeDtypeStruct((NC, N, D), jnp.float32),
                jax.ShapeDtypeStruct((NW * N,), jnp.float32)),
      mesh=mesh,
      scratch_types=(
          pltpu.VMEM((NBUF, CE, D), jnp.float32),
          pltpu.VMEM((NBUF, CE), jnp.int32),
          pltpu.VMEM((N,), jnp.float32),
          pltpu.VMEM_SHARED((N, D), jnp.float32),
          pltpu.SemaphoreType.DMA,
          pltpu.SemaphoreType.DMA,
          pltpu.SemaphoreType.DMA,
          pltpu.SemaphoreType.DMA,
          pltpu.SemaphoreType.DMA,
      ),
      compiler_params=pltpu.CompilerParams(needs_layout_passes=False),
  )
  return f(h_edge, edge_index, z640, z1d, history_deg, history_neigh)


_R = 2000


def _tc_combine_body(hs, fr, cr, ws, wn, b, rst_o, hist_o):
  hist = fr[0] + fr[1]
  deg = jnp.sum(cr[...], axis=1, keepdims=True) + 1.0
  h_neigh = hist * (1.0 / deg)
  rst = jnp.dot(hs[...], ws[...], preferred_element_type=jnp.float32)
  rst = rst + jnp.dot(h_neigh, wn[...], preferred_element_type=jnp.float32)
  rst_o[...] = rst + b[...]
  hist_o[...] = hist


def _tc_combine(h_self, feat, cntT, Wst, Wnt, bias):
  return pl.pallas_call(
      _tc_combine_body,
      grid=(N // _R,),
      in_specs=[
          pl.BlockSpec((_R, D), lambda i: (i, 0)),
          pl.BlockSpec((NC, _R, D), lambda i: (0, i, 0)),
          pl.BlockSpec((_R, NW), lambda i: (i, 0)),
          pl.BlockSpec((D, D), lambda i: (0, 0)),
          pl.BlockSpec((D, D), lambda i: (0, 0)),
          pl.BlockSpec((1, D), lambda i: (0, 0)),
      ],
      out_specs=[
          pl.BlockSpec((_R, D), lambda i: (i, 0)),
          pl.BlockSpec((_R, D), lambda i: (i, 0)),
      ],
      out_shape=[
          jax.ShapeDtypeStruct((N, D), jnp.float32),
          jax.ShapeDtypeStruct((N, D), jnp.float32),
      ],
      compiler_params=pltpu.CompilerParams(
          allow_input_fusion=[False, False, True, False, False, False]),
  )(h_self, feat, cntT, Wst, Wnt, bias)


def kernel(h_self, h_edge, history_neigh, history_deg, edge_index,
           W_self, b_self, W_neigh, b_neigh):
  ei = edge_index.astype(jnp.int32)
  z640 = jnp.zeros((RPT, D), jnp.float32)
  z1d = jnp.zeros((N,), jnp.float32)
  feat, cnt = _sc_scatter(h_edge, ei, z640, z1d, history_deg, history_neigh)
  cntT = cnt.reshape(NW, N).T
  bias = (b_self + b_neigh)[None, :]
  rst, hist = _tc_combine(h_self, feat, cntT, W_self.T, W_neigh.T, bias)
  return (rst, hist)


# speedup vs baseline: 1.0041x; 1.0041x over previous
import jax
import jax.numpy as jnp
from jax import lax
from jax.experimental import pallas as pl
from jax.experimental.pallas import tpu as pltpu
from jax.experimental.pallas import tpu_sc as plsc

N = 10000
E = 320000
D = 128

NC = 2
NS = 16
NW = NC * NS
CE = 128
NCHUNK = E // CE
MAXIT = -(-NCHUNK // NW)
NBUF = 2
PAIRS = -(-MAXIT // NBUF)
RPT = 640
RPT_LAST = N - (NS - 1) * RPT
ZRPT = N // NS


def _sc_scatter(h_edge, edge_index, z640, z1d, history_deg, history_neigh):
  mesh = plsc.VectorSubcoreMesh(core_axis_name="c", subcore_axis_name="s")

  def body(h_edge_hbm, ei_hbm, z640_hbm, z1d_hbm, hd_hbm, hn_hbm,
           feat_out, cnt_out,
           stage, idxbuf, cnt_local, acc_feat,
           fsem0, fsem1, isem0, isem1, ssem):
    c = lax.axis_index("c")
    s = lax.axis_index("s")
    wid = c * NS + s
    my_rows = jnp.where(s == NS - 1, RPT_LAST, RPT)
    sbase = s * RPT
    fsems = (fsem0, fsem1)
    isems = (isem0, isem1)

    @pl.when(c == 0)
    def _():
      pltpu.sync_copy(hn_hbm.at[pl.ds(sbase, my_rows)],
                      acc_feat.at[pl.ds(sbase, my_rows)])

    @pl.when(c != 0)
    def _():
      pltpu.sync_copy(z640_hbm.at[pl.ds(0, my_rows)],
                      acc_feat.at[pl.ds(sbase, my_rows)])

    @pl.when(wid == 0)
    def _():
      pltpu.sync_copy(hd_hbm, cnt_local)

    @pl.when(wid != 0)
    def _():
      pltpu.sync_copy(z1d_hbm, cnt_local)

    ones_vec = jnp.ones((16,), jnp.float32)

    def issue(it, b):
      k = wid + it * NW

      @pl.when(k < NCHUNK)
      def _():
        pltpu.async_copy(h_edge_hbm.at[pl.ds(k * CE, CE)], stage.at[b],
                         fsems[b])
        pltpu.async_copy(ei_hbm.at[1, pl.ds(k * CE, CE)], idxbuf.at[b],
                         isems[b])

    for b in range(NBUF):
      issue(b, b)

    plsc.subcore_barrier()

    def wait_scatter(it, b):
      k = wid + it * NW

      @pl.when(k < NCHUNK)
      def _():
        pltpu.make_async_copy(h_edge_hbm.at[pl.ds(0, CE)], stage.at[b],
                              fsems[b]).wait()
        pltpu.make_async_copy(ei_hbm.at[1, pl.ds(0, CE)], idxbuf.at[b],
                              isems[b]).wait()
        cd = pltpu.async_copy(stage.at[b], acc_feat.at[idxbuf.at[b]], ssem,
                              add=True)
        for v in range(CE // 16):
          idx16 = idxbuf[b, pl.ds(v * 16, 16)]
          plsc.addupdate_scatter(cnt_local, [idx16], ones_vec)
        cd.wait()

    def pair_body(i, carry):
      for b in range(NBUF):
        it = i * NBUF + b
        wait_scatter(it, b)
        issue(it + NBUF, b)
      return carry

    lax.fori_loop(0, PAIRS, pair_body, 0)
    plsc.subcore_barrier()

    cd1 = pltpu.async_copy(acc_feat.at[pl.ds(sbase, my_rows)],
                           feat_out.at[c, pl.ds(sbase, my_rows)], fsem0)
    cd2 = pltpu.async_copy(cnt_local, cnt_out.at[pl.ds(wid * N, N)], isem0)
    cd1.wait()
    cd2.wait()

  f = pl.kernel(
      body,
      out_type=(jax.ShapeDtypeStruct((NC, N, D), jnp.float32),
                jax.ShapeDtypeStruct((NW * N,), jnp.float32)),
      mesh=mesh,
      scratch_types=(
          pltpu.VMEM((NBUF, CE, D), jnp.float32),
          pltpu.VMEM((NBUF, CE), jnp.int32),
          pltpu.VMEM((N,), jnp.float32),
          pltpu.VMEM_SHARED((N, D), jnp.float32),
          pltpu.SemaphoreType.DMA,
          pltpu.SemaphoreType.DMA,
          pltpu.SemaphoreType.DMA,
          pltpu.SemaphoreType.DMA,
          pltpu.SemaphoreType.DMA,
      ),
      compiler_params=pltpu.CompilerParams(needs_layout_passes=False),
  )
  return f(h_edge, edge_index, z640, z1d, history_deg, history_neigh)


_R = 2000


def _tc_combine_body(hs, fr, cr, ws, wn, b, rst_o, hist_o):
  hist = fr[0] + fr[1]
  deg = jnp.sum(cr[...], axis=1, keepdims=True) + 1.0
  h_neigh = hist * (1.0 / deg)
  rst = jnp.dot(hs[...], ws[...], preferred_element_type=jnp.float32)
  rst = rst + jnp.dot(h_neigh, wn[...], preferred_element_type=jnp.float32)
  rst_o[...] = rst + b[...]
  hist_o[...] = hist


def _tc_combine(h_self, feat, cntT, Wst, Wnt, bias):
  return pl.pallas_call(
      _tc_combine_body,
      grid=(N // _R,),
      in_specs=[
          pl.BlockSpec((_R, D), lambda i: (i, 0)),
          pl.BlockSpec((NC, _R, D), lambda i: (0, i, 0)),
          pl.BlockSpec((_R, NW), lambda i: (i, 0)),
          pl.BlockSpec((D, D), lambda i: (0, 0)),
          pl.BlockSpec((D, D), lambda i: (0, 0)),
          pl.BlockSpec((1, D), lambda i: (0, 0)),
      ],
      out_specs=[
          pl.BlockSpec((_R, D), lambda i: (i, 0)),
          pl.BlockSpec((_R, D), lambda i: (i, 0)),
      ],
      out_shape=[
          jax.ShapeDtypeStruct((N, D), jnp.float32),
          jax.ShapeDtypeStruct((N, D), jnp.float32),
      ],
  )(h_self, feat, cntT, Wst, Wnt, bias)


def kernel(h_self, h_edge, history_neigh, history_deg, edge_index,
           W_self, b_self, W_neigh, b_neigh):
  ei = edge_index.astype(jnp.int32)
  z640 = jnp.zeros((RPT, D), jnp.float32)
  z1d = jnp.zeros((N,), jnp.float32)
  feat, cnt = _sc_scatter(h_edge, ei, z640, z1d, history_deg, history_neigh)
  cntT = cnt.reshape(NW, N).T
  bias = (b_self + b_neigh)[None, :]
  rst, hist = _tc_combine(h_self, feat, cntT, W_self.T, W_neigh.T, bias)
  return (rst, hist)
